# Initial kernel scaffold; baseline (speedup 1.0000x reference)
import jax
import jax.numpy as jnp
from jax.experimental import pallas as pl


def kernel(input, target, meter):
    raise NotImplementedError("write your pallas kernel here")


# speedup vs baseline: 22.2037x; 22.2037x over previous
import jax
import jax.numpy as jnp
from jax.experimental import pallas as pl
from jax.experimental.pallas import tpu as pltpu

_B = 64
_C = 80
_D = 256
_P = _B * (_B - 1) // 2


def _cosine_body(x_ref, out_ref, dist_ref):
    x = x_ref[...]
    gram = jax.lax.dot_general(
        x, x,
        dimension_numbers=(((2,), (2,)), ((1,), (1,))),
        preferred_element_type=jnp.float32,
    )
    row = jax.lax.broadcasted_iota(jnp.int32, (_C, _B, _B), 1)
    col = jax.lax.broadcasted_iota(jnp.int32, (_C, _B, _B), 2)
    diag = jnp.sum(jnp.where(row == col, gram, 0.0), axis=2)
    norm = jnp.sqrt(diag)
    denom = jnp.maximum(norm[:, :, None] * norm[:, None, :], 1e-9)
    dist = gram / denom
    dist_ref[...] = jnp.transpose(dist, (1, 2, 0))

    def body(k, carry):
        i = 62 - k
        start = 62 * i - (i * (i - 1)) // 2
        blk = dist_ref[pl.ds(i, 1), pl.ds(1, _B - 1), :]
        out_ref[pl.ds(start, _B - 1), :] = blk.reshape(_B - 1, _C)
        return carry

    jax.lax.fori_loop(0, _B - 1, body, 0)


def kernel(input, target, meter):
    del target, meter
    return pl.pallas_call(
        _cosine_body,
        out_shape=jax.ShapeDtypeStruct((_P, _C), jnp.float32),
        scratch_shapes=[pltpu.VMEM((_B, _B, _C), jnp.float32)],
    )(input)


# speedup vs baseline: 34.9693x; 1.5749x over previous
import jax
import jax.numpy as jnp
from jax.experimental import pallas as pl
from jax.experimental.pallas import tpu as pltpu

_B = 64
_C = 80
_D = 256
_P = _B * (_B - 1) // 2
_G = _C // 2


def _cosine_body(x_ref, out_ref):
    x = x_ref[...]
    xt = jnp.transpose(x, (1, 0, 2))
    a = jnp.concatenate([xt[:_G], xt[_G:]], axis=1)
    gram2 = jax.lax.dot_general(
        a, a,
        dimension_numbers=(((2,), (2,)), ((0,), (0,))),
        preferred_element_type=jnp.float32,
    )
    row = jax.lax.broadcasted_iota(jnp.int32, (_G, 2 * _B, 2 * _B), 1)
    col = jax.lax.broadcasted_iota(jnp.int32, (_G, 2 * _B, 2 * _B), 2)
    diag = jnp.sum(jnp.where(row == col, gram2, 0.0), axis=2)
    norm = jnp.sqrt(diag)
    denom = jnp.maximum(norm[:, :, None] * norm[:, None, :], 1e-9)
    dist2 = gram2 / denom
    dist = jnp.concatenate(
        [dist2[:, :_B, :_B], dist2[:, _B:, _B:]], axis=0)
    dist_t = jnp.transpose(dist, (1, 2, 0))

    for i in range(_B - 2, -1, -1):
        start = 62 * i - (i * (i - 1)) // 2
        blk = jax.lax.slice(dist_t, (i, 1, 0), (i + 1, _B, _C))
        out_ref[start:start + _B - 1, :] = blk.reshape(_B - 1, _C)


def kernel(input, target, meter):
    del target, meter
    return pl.pallas_call(
        _cosine_body,
        out_shape=jax.ShapeDtypeStruct((_P, _C), jnp.float32),
    )(input)
